# Initial kernel scaffold; baseline (speedup 1.0000x reference)
import jax
import jax.numpy as jnp
from jax.experimental import pallas as pl


def kernel(query_x, query_coord, input_x, input_coord, W_q, W_k, W_v, sh_mix, rad_w1, rad_b1, rad_w2, rad_b2, alpha_w, W_proj, b_proj, W_skip, b_skip):
    raise NotImplementedError("write your pallas kernel here")


# speedup vs baseline: 3.9094x; 3.9094x over previous
import functools

import jax
import jax.numpy as jnp
import numpy as np
from jax.experimental import pallas as pl
from jax.experimental.pallas import tpu as pltpu
from jax.experimental.pallas import tpu_sc as plsc

NQ = 4096; NS = 8192; K = 32; D = 256; H = 8; DH = D // H
LEN_DIM = 64; FC = 64; HALF = LEN_DIM // 2
R_MAX = 1.2; R_MIN = 0.05
E = NQ * K
TW = 2 * D + 128
TQ_TOPK = 256
TQ_MSG = 64


def _table_body(x_ref, c_ref, wk_ref, wv_ref, out_ref):
    x = x_ref[...]
    out_ref[:, 0:D] = jnp.dot(x, wk_ref[...], preferred_element_type=jnp.float32)
    out_ref[:, D:2 * D] = jnp.dot(x, wv_ref[...], preferred_element_type=jnp.float32)
    out_ref[:, 2 * D:TW] = c_ref[...]


def _build_table(input_x, coord128, W_k, W_v):
    RT = 512
    return pl.pallas_call(
        _table_body,
        grid=(NS // RT,),
        in_specs=[
            pl.BlockSpec((RT, D), lambda i: (i, 0)),
            pl.BlockSpec((RT, 128), lambda i: (i, 0)),
            pl.BlockSpec((D, D), lambda i: (0, 0)),
            pl.BlockSpec((D, D), lambda i: (0, 0)),
        ],
        out_specs=pl.BlockSpec((RT, TW), lambda i: (i, 0)),
        out_shape=jax.ShapeDtypeStruct((NS, TW), jnp.float32),
    )(input_x, coord128, W_k, W_v)


def _topk_body(qp_ref, st_ref, idx_ref):
    qp = qp_ref[...]
    st = st_ref[...]
    d2 = jnp.zeros((TQ_TOPK, NS), jnp.float32)
    for c in range(3):
        dc = qp[:, c:c + 1] - st[c:c + 1, :]
        d2 = d2 + dc * dc
    iota = jax.lax.broadcasted_iota(jnp.int32, (TQ_TOPK, NS), 1)
    for j in range(K):
        m = jnp.min(d2, axis=1, keepdims=True)
        cand = jnp.where(d2 <= m, iota, NS)
        ix = jnp.min(cand, axis=1, keepdims=True)
        idx_ref[:, j:j + 1] = ix
        d2 = jnp.where(iota == ix, jnp.float32(np.inf), d2)


def _topk(qpad8, st8):
    return pl.pallas_call(
        _topk_body,
        grid=(NQ // TQ_TOPK,),
        in_specs=[
            pl.BlockSpec((TQ_TOPK, 8), lambda i: (i, 0)),
            pl.BlockSpec((8, NS), lambda i: (0, 0)),
        ],
        out_specs=pl.BlockSpec((TQ_TOPK, K), lambda i: (i, 0)),
        out_shape=jax.ShapeDtypeStruct((NQ, K), jnp.int32),
    )(qpad8, st8)


_NW = 32
_PER_W = E // _NW
_CH = 64
_NCH = _PER_W // _CH


def _gather_sc(table, idx_flat):
    mesh = plsc.VectorSubcoreMesh(core_axis_name="c", subcore_axis_name="s")

    @functools.partial(
        pl.kernel, mesh=mesh,
        out_type=jax.ShapeDtypeStruct((E, TW), jnp.float32),
        scratch_types=[
            pltpu.VMEM((_CH,), jnp.int32),
            pltpu.VMEM((_CH, TW), jnp.float32),
            pltpu.SemaphoreType.DMA,
        ],
    )
    def k(table_hbm, idx_hbm, out_hbm, idx_v, rows_v, sem):
        wid = jax.lax.axis_index("s") * 2 + jax.lax.axis_index("c")
        base = wid * _PER_W

        @pl.loop(0, _NCH)
        def _(ci):
            b = base + ci * _CH
            pltpu.sync_copy(idx_hbm.at[pl.ds(b, _CH)], idx_v)
            pltpu.async_copy(table_hbm.at[idx_v], rows_v, sem).wait()
            pltpu.sync_copy(rows_v, out_hbm.at[pl.ds(b, _CH)])

    return k(table, idx_flat)


_SH_C1 = 0.4886025119029199
_SH_C2 = 1.0925484305920792
_SH_C20 = 0.31539156525252005
_SH_C22 = 0.5462742152960396


def _msg_body(g_ref, qx_ref, qc_ref, fr_ref, wq_ref, wskip_ref, shmix_ref,
              rw1s_ref, rw1c_ref, rb1_ref, rw2_ref, rb2_ref, aflat_ref,
              shead_ref, sheadt_ref, wproj_ref, bproj_ref, bskip_ref, out_ref):
    ET = TQ_MSG * K
    g = g_ref[...].reshape(ET, TW)
    xk = g[:, 0:D]
    xv = g[:, D:2 * D]
    sc = g[:, 2 * D:2 * D + 16]
    qc = qc_ref[...]
    qce = jnp.broadcast_to(qc.reshape(TQ_MSG, 1, 16), (TQ_MSG, K, 16)).reshape(ET, 16)
    ev = qce - sc
    len2 = jnp.sum(ev * ev, axis=1, keepdims=True)
    elen = jnp.sqrt(len2 + 1e-12)
    unit = ev / (elen + 1e-12)
    x = unit[:, 0:1]; y = unit[:, 1:2]; z = unit[:, 2:3]
    li = jax.lax.broadcasted_iota(jnp.int32, (ET, 16), 1)
    one = jnp.ones_like(x)
    comps = (0.28209479177387814 * one, _SH_C1 * y, _SH_C1 * z, _SH_C1 * x,
             _SH_C2 * x * y, _SH_C2 * y * z, _SH_C20 * (3.0 * z * z - 1.0),
             _SH_C2 * x * z, _SH_C22 * (x * x - y * y))
    sh = jnp.zeros((ET, 16), jnp.float32)
    for i, comp in enumerate(comps):
        sh = sh + jnp.where(li == i, comp, 0.0)
    w_max = jax.nn.sigmoid((R_MAX - elen) / (0.1 * R_MAX))
    w_min = jax.nn.sigmoid((elen - R_MIN) / (0.1 * R_MAX))
    log_w = jnp.log(w_max * w_min + 1e-12)
    ang = elen * fr_ref[...]
    hpre = (jnp.dot(jnp.sin(ang), rw1s_ref[...], preferred_element_type=jnp.float32)
            + jnp.dot(jnp.cos(ang), rw1c_ref[...], preferred_element_type=jnp.float32)
            + rb1_ref[...])
    hrad = hpre * jax.nn.sigmoid(hpre)
    rad = jnp.dot(hrad, rw2_ref[...], preferred_element_type=jnp.float32) + rb2_ref[...]
    sh_term = jnp.dot(sh, shmix_ref[...], preferred_element_type=jnp.float32)
    key_m = (xk + sh_term) * rad
    val_m = (xv + sh_term) * rad
    qx = qx_ref[...]
    q = jnp.dot(qx, wq_ref[...], preferred_element_type=jnp.float32)
    qe = jnp.broadcast_to(q.reshape(TQ_MSG, 1, D), (TQ_MSG, K, D)).reshape(ET, D)
    pre = qe + key_m
    pre = jnp.where(pre >= 0.0, pre, 0.2 * pre)
    lg = (jnp.dot(pre * aflat_ref[...], shead_ref[...],
                  preferred_element_type=jnp.float32) + log_w)
    lg3 = lg.reshape(TQ_MSG, K, H)
    mx = jnp.max(lg3, axis=1, keepdims=True)
    ex = jnp.exp(lg3 - mx)
    den = jnp.sum(ex, axis=1, keepdims=True)
    al = (ex / den).reshape(ET, H)
    alexp = jnp.dot(al, sheadt_ref[...], preferred_element_type=jnp.float32)
    agg = jnp.sum((val_m * alexp).reshape(TQ_MSG, K, D), axis=1)
    out = (jnp.dot(agg, wproj_ref[...], preferred_element_type=jnp.float32)
           + bproj_ref[...]
           + jnp.dot(qx, wskip_ref[...], preferred_element_type=jnp.float32)
           + bskip_ref[...])
    out_ref[...] = out


def _msg(g3, query_x, qc16, fr, W_q, W_skip, shmix16, rw1s, rw1c, rb1, rad_w2,
         rb2, aflat, shead, sheadt, W_proj, bproj, bskip):
    bc = lambda shape: pl.BlockSpec(shape, lambda i: tuple(0 for _ in shape))
    return pl.pallas_call(
        _msg_body,
        grid=(NQ // TQ_MSG,),
        in_specs=[
            pl.BlockSpec((TQ_MSG, K, TW), lambda i: (i, 0, 0)),
            pl.BlockSpec((TQ_MSG, D), lambda i: (i, 0)),
            pl.BlockSpec((TQ_MSG, 16), lambda i: (i, 0)),
            bc((1, HALF)),
            bc((D, D)),
            bc((D, D)),
            bc((16, D)),
            bc((HALF, FC)),
            bc((HALF, FC)),
            bc((1, FC)),
            bc((FC, D)),
            bc((1, D)),
            bc((1, D)),
            bc((D, H)),
            bc((H, D)),
            bc((D, D)),
            bc((1, D)),
            bc((1, D)),
        ],
        out_specs=pl.BlockSpec((TQ_MSG, D), lambda i: (i, 0)),
        out_shape=jax.ShapeDtypeStruct((NQ, D), jnp.float32),
    )(g3, query_x, qc16, fr, W_q, W_skip, shmix16, rw1s, rw1c, rb1, rad_w2,
      rb2, aflat, shead, sheadt, W_proj, bproj, bskip)


def kernel(query_x, query_coord, input_x, input_coord, W_q, W_k, W_v, sh_mix,
           rad_w1, rad_b1, rad_w2, rad_b2, alpha_w, W_proj, b_proj, W_skip,
           b_skip):
    f32 = jnp.float32
    qc16 = jnp.pad(query_coord, ((0, 0), (0, 13)))
    sc128 = jnp.pad(input_coord, ((0, 0), (0, 125)))
    qpad8 = jnp.pad(query_coord, ((0, 0), (0, 5)))
    st8 = jnp.pad(input_coord.T, ((0, 5), (0, 0)))
    fr = np.exp(-np.arange(HALF) * (np.log(10000.0) / (HALF - 1)))
    fr = jnp.asarray(fr, f32).reshape(1, HALF)
    shmix16 = jnp.pad(sh_mix, ((0, 7), (0, 0)))
    rw1s = rad_w1[:HALF]
    rw1c = rad_w1[HALF:]
    aflat = alpha_w.reshape(1, D)
    shead = jnp.asarray(np.kron(np.eye(H), np.ones((DH, 1))), f32)
    sheadt = jnp.asarray(np.kron(np.eye(H), np.ones((1, DH))), f32)

    table = _build_table(input_x, sc128, W_k, W_v)
    idx = _topk(qpad8, st8)
    g = _gather_sc(table, idx.reshape(E))
    out = _msg(g.reshape(NQ, K, TW), query_x, qc16, fr, W_q, W_skip, shmix16,
               rw1s, rw1c, rad_b1.reshape(1, FC), rad_w2, rad_b2.reshape(1, D),
               aflat, shead, sheadt, W_proj, b_proj.reshape(1, D),
               b_skip.reshape(1, D))
    return out
